# baseline (speedup 1.0000x reference)
import functools

import jax
import jax.numpy as jnp
from jax import lax
from jax.experimental import pallas as pl
from jax.experimental.pallas import tpu as pltpu
from jax.experimental.pallas import tpu_sc as plsc

_N_NODES = 10000
_D_FEAT = 128
_FLAT = _N_NODES * _D_FEAT
_NC = 2
_NS = 16
_NW = _NC * _NS
_PER_W = _FLAT // _NW

_mesh = plsc.VectorSubcoreMesh(core_axis_name="c", subcore_axis_name="s")


@functools.partial(
    pl.kernel,
    mesh=_mesh,
    out_type=jax.ShapeDtypeStruct((_FLAT,), jnp.float32),
    scratch_types=[pltpu.VMEM((_PER_W,), jnp.float32)],
)
def _sc_copy(x_hbm, o_hbm, buf):
    wid = lax.axis_index("s") * _NC + lax.axis_index("c")
    base = wid * _PER_W
    pltpu.sync_copy(x_hbm.at[pl.ds(base, _PER_W)], buf)
    pltpu.sync_copy(buf, o_hbm.at[pl.ds(base, _PER_W)])


def kernel(edge_index, r_2, batch, x):
    return _sc_copy(x.reshape(_FLAT)).reshape(_N_NODES, _D_FEAT)


# speedup vs baseline: 5.6744x; 5.6744x over previous
import jax
import jax.numpy as jnp
from jax.experimental import pallas as pl
from jax.experimental.pallas import tpu as pltpu

_N_NODES = 10000
_D_FEAT = 128
_K = 5
_CH = 2000


def _copy_body(x_ref, o_ref, buf, in_sem, out_sem):
    for i in range(_K):
        pltpu.make_async_copy(
            x_ref.at[pl.ds(i * _CH, _CH), :], buf.at[i], in_sem.at[i]
        ).start()
    for i in range(_K):
        pltpu.make_async_copy(
            x_ref.at[pl.ds(i * _CH, _CH), :], buf.at[i], in_sem.at[i]
        ).wait()
        pltpu.make_async_copy(
            buf.at[i], o_ref.at[pl.ds(i * _CH, _CH), :], out_sem.at[i]
        ).start()
    for i in range(_K):
        pltpu.make_async_copy(
            buf.at[i], o_ref.at[pl.ds(i * _CH, _CH), :], out_sem.at[i]
        ).wait()


def kernel(edge_index, r_2, batch, x):
    return pl.pallas_call(
        _copy_body,
        out_shape=jax.ShapeDtypeStruct((_N_NODES, _D_FEAT), jnp.float32),
        in_specs=[pl.BlockSpec(memory_space=pl.ANY)],
        out_specs=pl.BlockSpec(memory_space=pl.ANY),
        scratch_shapes=[
            pltpu.MemorySpace.VMEM((_K, _CH, _D_FEAT), jnp.float32),
            pltpu.SemaphoreType.DMA((_K,)),
            pltpu.SemaphoreType.DMA((_K,)),
        ],
    )(x)


# speedup vs baseline: 5.6916x; 1.0030x over previous
import jax
import jax.numpy as jnp
from jax.experimental import pallas as pl
from jax.experimental.pallas import tpu as pltpu

_N_NODES = 10000
_D_FEAT = 128
_CHUNKS = ((0, 200), (200, 2400), (2600, 2400), (5000, 2400),
           (7400, 2400), (9800, 200))
_K = len(_CHUNKS)


def _copy_body(x_ref, o_ref, buf, in_sem, out_sem):
    for i, (s, n) in enumerate(_CHUNKS):
        pltpu.make_async_copy(
            x_ref.at[pl.ds(s, n), :], buf.at[pl.ds(s, n), :], in_sem.at[i]
        ).start()
    for i, (s, n) in enumerate(_CHUNKS):
        pltpu.make_async_copy(
            x_ref.at[pl.ds(s, n), :], buf.at[pl.ds(s, n), :], in_sem.at[i]
        ).wait()
        pltpu.make_async_copy(
            buf.at[pl.ds(s, n), :], o_ref.at[pl.ds(s, n), :], out_sem.at[i]
        ).start()
    for i, (s, n) in enumerate(_CHUNKS):
        pltpu.make_async_copy(
            buf.at[pl.ds(s, n), :], o_ref.at[pl.ds(s, n), :], out_sem.at[i]
        ).wait()


def kernel(edge_index, r_2, batch, x):
    return pl.pallas_call(
        _copy_body,
        out_shape=jax.ShapeDtypeStruct((_N_NODES, _D_FEAT), jnp.float32),
        in_specs=[pl.BlockSpec(memory_space=pl.ANY)],
        out_specs=pl.BlockSpec(memory_space=pl.ANY),
        scratch_shapes=[
            pltpu.MemorySpace.VMEM((_N_NODES, _D_FEAT), jnp.float32),
            pltpu.SemaphoreType.DMA((_K,)),
            pltpu.SemaphoreType.DMA((_K,)),
        ],
    )(x)
